# baseline (device time: 103524 ns/iter reference)
import jax
import jax.numpy as jnp
from jax import lax
from jax.experimental import pallas as pl
from jax.experimental.pallas import tpu as pltpu

N_DEV = 4


def kernel(dy, W):
    m, k = dy.shape
    d = W.shape[0]
    chunk = m // N_DEV
    n_rs = N_DEV - 1
    n_ag = N_DEV - 1
    n_hops = n_rs + n_ag

    def body(dy_ref, w_ref, out_ref, comm_ref, send_sems, recv_sems):
        my = lax.axis_index("i")
        left = (my - 1) % N_DEV
        right = (my + 1) % N_DEV

        barrier_sem = pltpu.get_barrier_semaphore()
        for nbr in (left, right):
            pl.semaphore_signal(
                barrier_sem, inc=1,
                device_id=(nbr,), device_id_type=pl.DeviceIdType.MESH,
            )
        pl.semaphore_wait(barrier_sem, 2)

        out_ref[...] = lax.dot_general(
            dy_ref[...], w_ref[...],
            dimension_numbers=(((1,), (1,)), ((), ())),
            preferred_element_type=jnp.float32,
        )

        for s in range(n_rs):
            sc = (my - s) % N_DEV
            rc = (my - s - 1) % N_DEV
            rdma = pltpu.make_async_remote_copy(
                src_ref=out_ref.at[pl.ds(sc * chunk, chunk), :],
                dst_ref=comm_ref.at[s],
                send_sem=send_sems.at[s],
                recv_sem=recv_sems.at[s],
                device_id=(right,),
                device_id_type=pl.DeviceIdType.MESH,
            )
            rdma.start()
            rdma.wait()
            out_ref[pl.ds(rc * chunk, chunk), :] = (
                out_ref[pl.ds(rc * chunk, chunk), :] + comm_ref[s]
            )

        for t in range(n_ag):
            h = n_rs + t
            sc = (my + 1 - t) % N_DEV
            rdma = pltpu.make_async_remote_copy(
                src_ref=out_ref.at[pl.ds(sc * chunk, chunk), :],
                dst_ref=out_ref.at[pl.ds(sc * chunk, chunk), :],
                send_sem=send_sems.at[h],
                recv_sem=recv_sems.at[h],
                device_id=(right,),
                device_id_type=pl.DeviceIdType.MESH,
            )
            rdma.start()
            rdma.wait()

    return pl.pallas_call(
        body,
        out_shape=jax.ShapeDtypeStruct((m, d), jnp.float32),
        in_specs=[
            pl.BlockSpec(memory_space=pltpu.VMEM),
            pl.BlockSpec(memory_space=pltpu.VMEM),
        ],
        out_specs=pl.BlockSpec(memory_space=pltpu.VMEM),
        scratch_shapes=[
            pltpu.VMEM((n_rs, chunk, d), jnp.float32),
            pltpu.SemaphoreType.DMA((n_hops,)),
            pltpu.SemaphoreType.DMA((n_hops,)),
        ],
        compiler_params=pltpu.CompilerParams(collective_id=0),
    )(dy, W)


# device time: 62863 ns/iter; 1.6468x vs baseline; 1.6468x over previous
import jax
import jax.numpy as jnp
from jax import lax
from jax.experimental import pallas as pl
from jax.experimental.pallas import tpu as pltpu

N_DEV = 4


def kernel(dy, W):
    m, k = dy.shape
    d = W.shape[0]
    chunk = m // N_DEV
    half = d // 2
    n_rs = N_DEV - 1
    n_hops = 2 * n_rs

    def body(dy_ref, w_ref, out_ref, commR, commL, ssR, rsR, ssL, rsL):
        my = lax.axis_index("i")
        left = (my - 1) % N_DEV
        right = (my + 1) % N_DEV

        barrier_sem = pltpu.get_barrier_semaphore()
        for nbr in (left, right):
            pl.semaphore_signal(
                barrier_sem, inc=1,
                device_id=(nbr,), device_id_type=pl.DeviceIdType.MESH,
            )

        def compute_chunk(c):
            out_ref[pl.ds(c * chunk, chunk), :] = lax.dot_general(
                dy_ref[pl.ds(c * chunk, chunk), :], w_ref[...],
                dimension_numbers=(((1,), (1,)), ((), ())),
                preferred_element_type=jnp.float32,
            )

        compute_chunk(my)
        pl.semaphore_wait(barrier_sem, 2)

        for s in range(n_rs):
            scR = (my - s) % N_DEV
            scL = (my + s) % N_DEV
            rR = pltpu.make_async_remote_copy(
                src_ref=out_ref.at[pl.ds(scR * chunk, chunk), pl.ds(0, half)],
                dst_ref=commR.at[s],
                send_sem=ssR.at[s], recv_sem=rsR.at[s],
                device_id=(right,), device_id_type=pl.DeviceIdType.MESH,
            )
            rL = pltpu.make_async_remote_copy(
                src_ref=out_ref.at[pl.ds(scL * chunk, chunk), pl.ds(half, half)],
                dst_ref=commL.at[s],
                send_sem=ssL.at[s], recv_sem=rsL.at[s],
                device_id=(left,), device_id_type=pl.DeviceIdType.MESH,
            )
            rR.start()
            rL.start()
            if s == 0:
                compute_chunk((my + 1) % N_DEV)
                compute_chunk((my - 1) % N_DEV)
            elif s == 1:
                compute_chunk((my + 2) % N_DEV)
            rcR = (my - s - 1) % N_DEV
            rcL = (my + s + 1) % N_DEV
            rR.wait()
            out_ref[pl.ds(rcR * chunk, chunk), pl.ds(0, half)] = (
                out_ref[pl.ds(rcR * chunk, chunk), pl.ds(0, half)] + commR[s]
            )
            rL.wait()
            out_ref[pl.ds(rcL * chunk, chunk), pl.ds(half, half)] = (
                out_ref[pl.ds(rcL * chunk, chunk), pl.ds(half, half)] + commL[s]
            )

        for t in range(n_rs):
            h = n_rs + t
            scR = (my + 1 - t) % N_DEV
            scL = (my - 1 + t) % N_DEV
            rR = pltpu.make_async_remote_copy(
                src_ref=out_ref.at[pl.ds(scR * chunk, chunk), pl.ds(0, half)],
                dst_ref=out_ref.at[pl.ds(scR * chunk, chunk), pl.ds(0, half)],
                send_sem=ssR.at[h], recv_sem=rsR.at[h],
                device_id=(right,), device_id_type=pl.DeviceIdType.MESH,
            )
            rL = pltpu.make_async_remote_copy(
                src_ref=out_ref.at[pl.ds(scL * chunk, chunk), pl.ds(half, half)],
                dst_ref=out_ref.at[pl.ds(scL * chunk, chunk), pl.ds(half, half)],
                send_sem=ssL.at[h], recv_sem=rsL.at[h],
                device_id=(left,), device_id_type=pl.DeviceIdType.MESH,
            )
            rR.start()
            rL.start()
            rR.wait()
            rL.wait()

    return pl.pallas_call(
        body,
        out_shape=jax.ShapeDtypeStruct((m, d), jnp.float32),
        in_specs=[
            pl.BlockSpec(memory_space=pltpu.VMEM),
            pl.BlockSpec(memory_space=pltpu.VMEM),
        ],
        out_specs=pl.BlockSpec(memory_space=pltpu.VMEM),
        scratch_shapes=[
            pltpu.VMEM((n_rs, chunk, half), jnp.float32),
            pltpu.VMEM((n_rs, chunk, half), jnp.float32),
            pltpu.SemaphoreType.DMA((n_hops,)),
            pltpu.SemaphoreType.DMA((n_hops,)),
            pltpu.SemaphoreType.DMA((n_hops,)),
            pltpu.SemaphoreType.DMA((n_hops,)),
        ],
        compiler_params=pltpu.CompilerParams(collective_id=0),
    )(dy, W)


# device time: 55165 ns/iter; 1.8766x vs baseline; 1.1395x over previous
import jax
import jax.numpy as jnp
from jax import lax
from jax.experimental import pallas as pl
from jax.experimental.pallas import tpu as pltpu

N_DEV = 4
N_RS = N_DEV - 1
N_HOPS = 2 * N_RS
NQ = 2


def kernel(dy, W):
    m, k = dy.shape
    d = W.shape[0]
    chunk = m // N_DEV
    half = d // 2
    qw = half // NQ

    def body(dy_ref, w_ref, out_ref, commR, commL, ssR, rsR, ssL, rsL):
        my = lax.axis_index("i")
        left = (my - 1) % N_DEV
        right = (my + 1) % N_DEV

        barrier_sem = pltpu.get_barrier_semaphore()
        for nbr in (left, right):
            pl.semaphore_signal(
                barrier_sem, inc=1,
                device_id=(nbr,), device_id_type=pl.DeviceIdType.MESH,
            )

        def compute(c, c0, w):
            out_ref[pl.ds(c * chunk, chunk), pl.ds(c0, w)] = lax.dot_general(
                dy_ref[pl.ds(c * chunk, chunk), :],
                w_ref[pl.ds(c0, w), :],
                dimension_numbers=(((1,), (1,)), ((), ())),
                preferred_element_type=jnp.float32,
            )

        pending = []

        def out_slice(c, col0):
            return out_ref.at[pl.ds(c * chunk, chunk), pl.ds(col0, qw)]

        def send_rs(dirn, s, j):
            if dirn == 0:
                r = pltpu.make_async_remote_copy(
                    src_ref=out_slice((my - s) % N_DEV, j * qw),
                    dst_ref=commR.at[s, j],
                    send_sem=ssR.at[s, j], recv_sem=rsR.at[s, j],
                    device_id=(right,), device_id_type=pl.DeviceIdType.MESH,
                )
            else:
                r = pltpu.make_async_remote_copy(
                    src_ref=out_slice((my + s) % N_DEV, half + j * qw),
                    dst_ref=commL.at[s, j],
                    send_sem=ssL.at[s, j], recv_sem=rsL.at[s, j],
                    device_id=(left,), device_id_type=pl.DeviceIdType.MESH,
                )
            r.start()
            pending.append(r)

        def send_ag(dirn, t, j):
            h = N_RS + t
            if dirn == 0:
                sl = out_slice((my + 1 - t) % N_DEV, j * qw)
                r = pltpu.make_async_remote_copy(
                    src_ref=sl, dst_ref=sl,
                    send_sem=ssR.at[h, j], recv_sem=rsR.at[h, j],
                    device_id=(right,), device_id_type=pl.DeviceIdType.MESH,
                )
            else:
                sl = out_slice((my - 1 + t) % N_DEV, half + j * qw)
                r = pltpu.make_async_remote_copy(
                    src_ref=sl, dst_ref=sl,
                    send_sem=ssL.at[h, j], recv_sem=rsL.at[h, j],
                    device_id=(left,), device_id_type=pl.DeviceIdType.MESH,
                )
            r.start()
            pending.append(r)

        def wait_recv(dst, sem):
            pltpu.make_async_remote_copy(
                src_ref=dst, dst_ref=dst, send_sem=sem, recv_sem=sem,
                device_id=(left,), device_id_type=pl.DeviceIdType.MESH,
            ).wait_recv()

        def recv_rs(dirn, s, j):
            if dirn == 0:
                wait_recv(commR.at[s, j], rsR.at[s, j])
                rc = (my - s - 1) % N_DEV
                out_ref[pl.ds(rc * chunk, chunk), pl.ds(j * qw, qw)] = (
                    out_ref[pl.ds(rc * chunk, chunk), pl.ds(j * qw, qw)]
                    + commR[s, j]
                )
            else:
                wait_recv(commL.at[s, j], rsL.at[s, j])
                rc = (my + s + 1) % N_DEV
                c0 = half + j * qw
                out_ref[pl.ds(rc * chunk, chunk), pl.ds(c0, qw)] = (
                    out_ref[pl.ds(rc * chunk, chunk), pl.ds(c0, qw)]
                    + commL[s, j]
                )

        def recv_ag(dirn, t, j):
            h = N_RS + t
            if dirn == 0:
                wait_recv(out_slice((my - t) % N_DEV, j * qw), rsR.at[h, j])
            else:
                wait_recv(
                    out_slice((my + t) % N_DEV, half + j * qw), rsL.at[h, j]
                )

        compute(my, 0, d)
        pl.semaphore_wait(barrier_sem, 2)

        for j in range(NQ):
            send_rs(0, 0, j)
        for j in range(NQ):
            send_rs(1, 0, j)

        compute((my - 1) % N_DEV, 0, half)
        compute((my + 1) % N_DEV, half, half)

        for j in range(NQ):
            recv_rs(0, 0, j)
            send_rs(0, 1, j)
        for j in range(NQ):
            recv_rs(1, 0, j)
            send_rs(1, 1, j)

        compute((my + 2) % N_DEV, 0, d)

        for j in range(NQ):
            recv_rs(0, 1, j)
            send_rs(0, 2, j)
        for j in range(NQ):
            recv_rs(1, 1, j)
            send_rs(1, 2, j)

        compute((my + 1) % N_DEV, 0, half)
        compute((my - 1) % N_DEV, half, half)

        for j in range(NQ):
            recv_rs(0, 2, j)
            send_ag(0, 0, j)
        for j in range(NQ):
            recv_rs(1, 2, j)
            send_ag(1, 0, j)

        for t in (1, 2):
            for j in range(NQ):
                recv_ag(0, t - 1, j)
                send_ag(0, t, j)
            for j in range(NQ):
                recv_ag(1, t - 1, j)
                send_ag(1, t, j)

        for j in range(NQ):
            recv_ag(0, 2, j)
            recv_ag(1, 2, j)
        for r in pending:
            r.wait_send()

    return pl.pallas_call(
        body,
        out_shape=jax.ShapeDtypeStruct((m, d), jnp.float32),
        in_specs=[
            pl.BlockSpec(memory_space=pltpu.VMEM),
            pl.BlockSpec(memory_space=pltpu.VMEM),
        ],
        out_specs=pl.BlockSpec(memory_space=pltpu.VMEM),
        scratch_shapes=[
            pltpu.VMEM((N_RS, NQ, chunk, half // NQ), jnp.float32),
            pltpu.VMEM((N_RS, NQ, chunk, half // NQ), jnp.float32),
            pltpu.SemaphoreType.DMA((N_HOPS, NQ)),
            pltpu.SemaphoreType.DMA((N_HOPS, NQ)),
            pltpu.SemaphoreType.DMA((N_HOPS, NQ)),
            pltpu.SemaphoreType.DMA((N_HOPS, NQ)),
        ],
        compiler_params=pltpu.CompilerParams(collective_id=0),
    )(dy, W)


# device time: 53863 ns/iter; 1.9220x vs baseline; 1.0242x over previous
import os

import jax
import jax.numpy as jnp
from jax import lax
from jax.experimental import pallas as pl
from jax.experimental.pallas import tpu as pltpu

_KMODE = os.environ.get("KMODE", "full")

N_DEV = 4
N_RS = N_DEV - 1
N_HOPS = 2 * N_RS
NQ = 2


def kernel(dy, W):
    m, k = dy.shape
    d = W.shape[0]
    chunk = m // N_DEV
    half = d // 2
    qw = half // NQ

    def body(dy_ref, w_ref, out_ref, commR, commL, ssR, rsR, ssL, rsL):
        my = lax.axis_index("i")
        left = (my - 1) % N_DEV
        right = (my + 1) % N_DEV

        if _KMODE != "gemm":
            barrier_sem = pltpu.get_barrier_semaphore()
            for nbr in (left, right):
                pl.semaphore_signal(
                    barrier_sem, inc=1,
                    device_id=(nbr,), device_id_type=pl.DeviceIdType.MESH,
                )

        def compute(c, c0, w):
            if _KMODE == "comm":
                return
            out_ref[pl.ds(c * chunk, chunk), pl.ds(c0, w)] = lax.dot_general(
                dy_ref[pl.ds(c * chunk, chunk), :],
                w_ref[pl.ds(c0, w), :],
                dimension_numbers=(((1,), (1,)), ((), ())),
                preferred_element_type=jnp.float32,
            )

        pending = []

        def out_slice(c, col0):
            return out_ref.at[pl.ds(c * chunk, chunk), pl.ds(col0, qw)]

        def send_rs(dirn, s, j):
            if dirn == 0:
                r = pltpu.make_async_remote_copy(
                    src_ref=out_slice((my - s) % N_DEV, j * qw),
                    dst_ref=commR.at[s, j],
                    send_sem=ssR.at[s, j], recv_sem=rsR.at[s, j],
                    device_id=(right,), device_id_type=pl.DeviceIdType.MESH,
                )
            else:
                r = pltpu.make_async_remote_copy(
                    src_ref=out_slice((my + s) % N_DEV, half + j * qw),
                    dst_ref=commL.at[s, j],
                    send_sem=ssL.at[s, j], recv_sem=rsL.at[s, j],
                    device_id=(left,), device_id_type=pl.DeviceIdType.MESH,
                )
            r.start()
            pending.append(r)

        def send_ag(dirn, t, j):
            h = N_RS + t
            if dirn == 0:
                sl = out_slice((my + 1 - t) % N_DEV, j * qw)
                r = pltpu.make_async_remote_copy(
                    src_ref=sl, dst_ref=sl,
                    send_sem=ssR.at[h, j], recv_sem=rsR.at[h, j],
                    device_id=(right,), device_id_type=pl.DeviceIdType.MESH,
                )
            else:
                sl = out_slice((my - 1 + t) % N_DEV, half + j * qw)
                r = pltpu.make_async_remote_copy(
                    src_ref=sl, dst_ref=sl,
                    send_sem=ssL.at[h, j], recv_sem=rsL.at[h, j],
                    device_id=(left,), device_id_type=pl.DeviceIdType.MESH,
                )
            r.start()
            pending.append(r)

        def wait_recv(dst, sem):
            pltpu.make_async_remote_copy(
                src_ref=dst, dst_ref=dst, send_sem=sem, recv_sem=sem,
                device_id=(left,), device_id_type=pl.DeviceIdType.MESH,
            ).wait_recv()

        def recv_rs(dirn, s, j):
            if dirn == 0:
                wait_recv(commR.at[s, j], rsR.at[s, j])
                rc = (my - s - 1) % N_DEV
                out_ref[pl.ds(rc * chunk, chunk), pl.ds(j * qw, qw)] = (
                    out_ref[pl.ds(rc * chunk, chunk), pl.ds(j * qw, qw)]
                    + commR[s, j]
                )
            else:
                wait_recv(commL.at[s, j], rsL.at[s, j])
                rc = (my + s + 1) % N_DEV
                c0 = half + j * qw
                out_ref[pl.ds(rc * chunk, chunk), pl.ds(c0, qw)] = (
                    out_ref[pl.ds(rc * chunk, chunk), pl.ds(c0, qw)]
                    + commL[s, j]
                )

        def recv_ag(dirn, t, j):
            h = N_RS + t
            if dirn == 0:
                wait_recv(out_slice((my - t) % N_DEV, j * qw), rsR.at[h, j])
            else:
                wait_recv(
                    out_slice((my + t) % N_DEV, half + j * qw), rsL.at[h, j]
                )

        if _KMODE == "gemm":
            for c in range(N_DEV):
                compute((my + c) % N_DEV, 0, d)
            return

        compute(my, 0, d)
        pl.semaphore_wait(barrier_sem, 2)

        for j in range(NQ):
            send_rs(0, 0, j)
        for j in range(NQ):
            send_rs(1, 0, j)

        compute((my - 1) % N_DEV, 0, half)
        compute((my + 1) % N_DEV, half, half)

        for j in range(NQ):
            recv_rs(0, 0, j)
            send_rs(0, 1, j)
        for j in range(NQ):
            recv_rs(1, 0, j)
            send_rs(1, 1, j)

        compute((my + 2) % N_DEV, 0, d)

        for j in range(NQ):
            recv_rs(0, 1, j)
            send_rs(0, 2, j)
        for j in range(NQ):
            recv_rs(1, 1, j)
            send_rs(1, 2, j)

        compute((my + 1) % N_DEV, 0, half)
        compute((my - 1) % N_DEV, half, half)

        for j in range(NQ):
            recv_rs(0, 2, j)
            send_ag(0, 0, j)
        for j in range(NQ):
            recv_rs(1, 2, j)
            send_ag(1, 0, j)

        for t in (1, 2):
            for j in range(NQ):
                recv_ag(0, t - 1, j)
                send_ag(0, t, j)
            for j in range(NQ):
                recv_ag(1, t - 1, j)
                send_ag(1, t, j)

        for j in range(NQ):
            recv_ag(0, 2, j)
            recv_ag(1, 2, j)
        for r in pending:
            r.wait_send()

    return pl.pallas_call(
        body,
        out_shape=jax.ShapeDtypeStruct((m, d), jnp.float32),
        in_specs=[
            pl.BlockSpec(memory_space=pltpu.VMEM),
            pl.BlockSpec(memory_space=pltpu.VMEM),
        ],
        out_specs=pl.BlockSpec(memory_space=pltpu.VMEM),
        scratch_shapes=[
            pltpu.VMEM((N_RS, NQ, chunk, half // NQ), jnp.float32),
            pltpu.VMEM((N_RS, NQ, chunk, half // NQ), jnp.float32),
            pltpu.SemaphoreType.DMA((N_HOPS, NQ)),
            pltpu.SemaphoreType.DMA((N_HOPS, NQ)),
            pltpu.SemaphoreType.DMA((N_HOPS, NQ)),
            pltpu.SemaphoreType.DMA((N_HOPS, NQ)),
        ],
        compiler_params=pltpu.CompilerParams(
            collective_id=None if _KMODE == "gemm" else 0
        ),
    )(dy, W)


# device time: 42532 ns/iter; 2.4340x vs baseline; 1.2664x over previous
import os

import jax
import jax.numpy as jnp
from jax import lax
from jax.experimental import pallas as pl
from jax.experimental.pallas import tpu as pltpu

_KMODE = os.environ.get("KMODE", "full")

N_DEV = 4
N_RS = N_DEV - 1
N_HOPS = 2 * N_RS
NQ = 2
WIRE_DT = jnp.bfloat16


def kernel(dy, W):
    m, k = dy.shape
    d = W.shape[0]
    chunk = m // N_DEV
    half = d // 2
    qw = half // NQ

    def body(
        dy_ref, w_ref, out_ref,
        sbufR, rbufR, agR, sbufL, rbufL, agL,
        ssR, rsR, ssL, rsL,
    ):
        my = lax.axis_index("i")
        left = (my - 1) % N_DEV
        right = (my + 1) % N_DEV

        if _KMODE != "gemm":
            barrier_sem = pltpu.get_barrier_semaphore()
            for nbr in (left, right):
                pl.semaphore_signal(
                    barrier_sem, inc=1,
                    device_id=(nbr,), device_id_type=pl.DeviceIdType.MESH,
                )

        def compute(c, c0, w):
            if _KMODE == "comm":
                return
            out_ref[pl.ds(c * chunk, chunk), pl.ds(c0, w)] = lax.dot_general(
                dy_ref[pl.ds(c * chunk, chunk), :],
                w_ref[pl.ds(c0, w), :],
                dimension_numbers=(((1,), (1,)), ((), ())),
                preferred_element_type=jnp.float32,
            )

        pending = []

        def rd(c, j, dirn):
            c0 = j * qw if dirn == 0 else half + j * qw
            return out_ref[pl.ds(c * chunk, chunk), pl.ds(c0, qw)]

        def wr(c, j, dirn, val):
            c0 = j * qw if dirn == 0 else half + j * qw
            out_ref[pl.ds(c * chunk, chunk), pl.ds(c0, qw)] = val

        def send(src, dst, send_sem, recv_sem, dev):
            r = pltpu.make_async_remote_copy(
                src_ref=src, dst_ref=dst, send_sem=send_sem,
                recv_sem=recv_sem, device_id=(dev,),
                device_id_type=pl.DeviceIdType.MESH,
            )
            r.start()
            pending.append(r)

        def wait_recv(dst, sem):
            pltpu.make_async_remote_copy(
                src_ref=dst, dst_ref=dst, send_sem=sem, recv_sem=sem,
                device_id=(left,), device_id_type=pl.DeviceIdType.MESH,
            ).wait_recv()

        def send_rs(dirn, s, j):
            if dirn == 0:
                sbufR[s, j] = rd((my - s) % N_DEV, j, 0).astype(WIRE_DT)
                send(sbufR.at[s, j], rbufR.at[s, j],
                     ssR.at[s, j], rsR.at[s, j], right)
            else:
                sbufL[s, j] = rd((my + s) % N_DEV, j, 1).astype(WIRE_DT)
                send(sbufL.at[s, j], rbufL.at[s, j],
                     ssL.at[s, j], rsL.at[s, j], left)

        def recv_rs(dirn, s, j):
            if dirn == 0:
                wait_recv(rbufR.at[s, j], rsR.at[s, j])
                rc = (my - s - 1) % N_DEV
                wr(rc, j, 0, rd(rc, j, 0) + rbufR[s, j].astype(jnp.float32))
            else:
                wait_recv(rbufL.at[s, j], rsL.at[s, j])
                rc = (my + s + 1) % N_DEV
                wr(rc, j, 1, rd(rc, j, 1) + rbufL[s, j].astype(jnp.float32))

        if _KMODE == "gemm":
            for c in range(N_DEV):
                compute((my + c) % N_DEV, 0, d)
            return

        compute(my, 0, d)
        pl.semaphore_wait(barrier_sem, 2)

        for j in range(NQ):
            send_rs(0, 0, j)
        for j in range(NQ):
            send_rs(1, 0, j)

        compute((my - 1) % N_DEV, 0, half)
        compute((my + 1) % N_DEV, half, half)

        for j in range(NQ):
            recv_rs(0, 0, j)
            send_rs(0, 1, j)
        for j in range(NQ):
            recv_rs(1, 0, j)
            send_rs(1, 1, j)

        compute((my + 2) % N_DEV, 0, d)

        for j in range(NQ):
            recv_rs(0, 1, j)
            send_rs(0, 2, j)
        for j in range(NQ):
            recv_rs(1, 1, j)
            send_rs(1, 2, j)

        compute((my + 1) % N_DEV, 0, half)
        compute((my - 1) % N_DEV, half, half)

        for j in range(NQ):
            recv_rs(0, 2, j)
            rc = (my + 1) % N_DEV
            agR[3, j] = rd(rc, j, 0).astype(WIRE_DT)
            send(agR.at[3, j], agR.at[0, j],
                 ssR.at[N_RS, j], rsR.at[N_RS, j], right)
        for j in range(NQ):
            recv_rs(1, 2, j)
            rc = (my - 1) % N_DEV
            agL[3, j] = rd(rc, j, 1).astype(WIRE_DT)
            send(agL.at[3, j], agL.at[0, j],
                 ssL.at[N_RS, j], rsL.at[N_RS, j], left)

        for t in (1, 2):
            h = N_RS + t
            for j in range(NQ):
                wait_recv(agR.at[t - 1, j], rsR.at[h - 1, j])
                send(agR.at[t - 1, j], agR.at[t, j],
                     ssR.at[h, j], rsR.at[h, j], right)
                wr((my - t + 1) % N_DEV, j, 0,
                   agR[t - 1, j].astype(jnp.float32))
            for j in range(NQ):
                wait_recv(agL.at[t - 1, j], rsL.at[h - 1, j])
                send(agL.at[t - 1, j], agL.at[t, j],
                     ssL.at[h, j], rsL.at[h, j], left)
                wr((my + t - 1) % N_DEV, j, 1,
                   agL[t - 1, j].astype(jnp.float32))

        for j in range(NQ):
            wait_recv(agR.at[2, j], rsR.at[N_HOPS - 1, j])
            wr((my - 2) % N_DEV, j, 0, agR[2, j].astype(jnp.float32))
            wait_recv(agL.at[2, j], rsL.at[N_HOPS - 1, j])
            wr((my + 2) % N_DEV, j, 1, agL[2, j].astype(jnp.float32))
        for r in pending:
            r.wait_send()

    return pl.pallas_call(
        body,
        out_shape=jax.ShapeDtypeStruct((m, d), jnp.float32),
        in_specs=[
            pl.BlockSpec(memory_space=pltpu.VMEM),
            pl.BlockSpec(memory_space=pltpu.VMEM),
        ],
        out_specs=pl.BlockSpec(memory_space=pltpu.VMEM),
        scratch_shapes=[
            pltpu.VMEM((N_RS, NQ, chunk, qw), WIRE_DT),
            pltpu.VMEM((N_RS, NQ, chunk, qw), WIRE_DT),
            pltpu.VMEM((N_DEV, NQ, chunk, qw), WIRE_DT),
            pltpu.VMEM((N_RS, NQ, chunk, qw), WIRE_DT),
            pltpu.VMEM((N_RS, NQ, chunk, qw), WIRE_DT),
            pltpu.VMEM((N_DEV, NQ, chunk, qw), WIRE_DT),
            pltpu.SemaphoreType.DMA((N_HOPS, NQ)),
            pltpu.SemaphoreType.DMA((N_HOPS, NQ)),
            pltpu.SemaphoreType.DMA((N_HOPS, NQ)),
            pltpu.SemaphoreType.DMA((N_HOPS, NQ)),
        ],
        compiler_params=pltpu.CompilerParams(
            collective_id=None if _KMODE == "gemm" else 0
        ),
    )(dy, W)


# device time: 42474 ns/iter; 2.4373x vs baseline; 1.0014x over previous
import os

import jax
import jax.numpy as jnp
from jax import lax
from jax.experimental import pallas as pl
from jax.experimental.pallas import tpu as pltpu

_KMODE = os.environ.get("KMODE", "full")

N_DEV = 4
N_RS = N_DEV - 1
N_HOPS = 2 * N_RS
NQ = 2
WIRE_DT = jnp.bfloat16


def kernel(dy, W):
    m, k = dy.shape
    d = W.shape[0]
    chunk = m // N_DEV
    half = d // 2
    qw = half // NQ

    def body(
        dy_ref, w_ref, out_ref,
        sbufR, rbufR, agR, sbufL, rbufL, agL,
        ssR, rsR, ssL, rsL,
    ):
        my = lax.axis_index("i")
        left = (my - 1) % N_DEV
        right = (my + 1) % N_DEV

        if _KMODE != "gemm":
            barrier_sem = pltpu.get_barrier_semaphore()
            for nbr in (left, right):
                pl.semaphore_signal(
                    barrier_sem, inc=1,
                    device_id=(nbr,), device_id_type=pl.DeviceIdType.MESH,
                )

        def compute(c, c0, w):
            if _KMODE == "comm":
                return
            out_ref[pl.ds(c * chunk, chunk), pl.ds(c0, w)] = lax.dot_general(
                dy_ref[pl.ds(c * chunk, chunk), :].astype(jnp.bfloat16),
                w_ref[pl.ds(c0, w), :].astype(jnp.bfloat16),
                dimension_numbers=(((1,), (1,)), ((), ())),
                preferred_element_type=jnp.float32,
            )

        pending = []

        def rd(c, j, dirn):
            c0 = j * qw if dirn == 0 else half + j * qw
            return out_ref[pl.ds(c * chunk, chunk), pl.ds(c0, qw)]

        def wr(c, j, dirn, val):
            c0 = j * qw if dirn == 0 else half + j * qw
            out_ref[pl.ds(c * chunk, chunk), pl.ds(c0, qw)] = val

        def send(src, dst, send_sem, recv_sem, dev):
            r = pltpu.make_async_remote_copy(
                src_ref=src, dst_ref=dst, send_sem=send_sem,
                recv_sem=recv_sem, device_id=(dev,),
                device_id_type=pl.DeviceIdType.MESH,
            )
            r.start()
            pending.append(r)

        def wait_recv(dst, sem):
            pltpu.make_async_remote_copy(
                src_ref=dst, dst_ref=dst, send_sem=sem, recv_sem=sem,
                device_id=(left,), device_id_type=pl.DeviceIdType.MESH,
            ).wait_recv()

        def send_rs(dirn, s, j):
            if dirn == 0:
                sbufR[s, j] = rd((my - s) % N_DEV, j, 0).astype(WIRE_DT)
                send(sbufR.at[s, j], rbufR.at[s, j],
                     ssR.at[s, j], rsR.at[s, j], right)
            else:
                sbufL[s, j] = rd((my + s) % N_DEV, j, 1).astype(WIRE_DT)
                send(sbufL.at[s, j], rbufL.at[s, j],
                     ssL.at[s, j], rsL.at[s, j], left)

        def recv_rs(dirn, s, j):
            if dirn == 0:
                wait_recv(rbufR.at[s, j], rsR.at[s, j])
                rc = (my - s - 1) % N_DEV
                wr(rc, j, 0, rd(rc, j, 0) + rbufR[s, j].astype(jnp.float32))
            else:
                wait_recv(rbufL.at[s, j], rsL.at[s, j])
                rc = (my + s + 1) % N_DEV
                wr(rc, j, 1, rd(rc, j, 1) + rbufL[s, j].astype(jnp.float32))

        if _KMODE == "gemm":
            for c in range(N_DEV):
                compute((my + c) % N_DEV, 0, d)
            return

        compute(my, 0, d)
        pl.semaphore_wait(barrier_sem, 2)

        for j in range(NQ):
            send_rs(0, 0, j)
        for j in range(NQ):
            send_rs(1, 0, j)

        compute((my - 1) % N_DEV, 0, half)
        compute((my + 1) % N_DEV, half, half)

        for j in range(NQ):
            recv_rs(0, 0, j)
            send_rs(0, 1, j)
        for j in range(NQ):
            recv_rs(1, 0, j)
            send_rs(1, 1, j)

        compute((my + 2) % N_DEV, 0, d)

        for j in range(NQ):
            recv_rs(0, 1, j)
            send_rs(0, 2, j)
        for j in range(NQ):
            recv_rs(1, 1, j)
            send_rs(1, 2, j)

        compute((my + 1) % N_DEV, 0, half)
        compute((my - 1) % N_DEV, half, half)

        for j in range(NQ):
            recv_rs(0, 2, j)
            rc = (my + 1) % N_DEV
            agR[3, j] = rd(rc, j, 0).astype(WIRE_DT)
            send(agR.at[3, j], agR.at[0, j],
                 ssR.at[N_RS, j], rsR.at[N_RS, j], right)
        for j in range(NQ):
            recv_rs(1, 2, j)
            rc = (my - 1) % N_DEV
            agL[3, j] = rd(rc, j, 1).astype(WIRE_DT)
            send(agL.at[3, j], agL.at[0, j],
                 ssL.at[N_RS, j], rsL.at[N_RS, j], left)

        for t in (1, 2):
            h = N_RS + t
            for j in range(NQ):
                wait_recv(agR.at[t - 1, j], rsR.at[h - 1, j])
                send(agR.at[t - 1, j], agR.at[t, j],
                     ssR.at[h, j], rsR.at[h, j], right)
                wr((my - t + 1) % N_DEV, j, 0,
                   agR[t - 1, j].astype(jnp.float32))
            for j in range(NQ):
                wait_recv(agL.at[t - 1, j], rsL.at[h - 1, j])
                send(agL.at[t - 1, j], agL.at[t, j],
                     ssL.at[h, j], rsL.at[h, j], left)
                wr((my + t - 1) % N_DEV, j, 1,
                   agL[t - 1, j].astype(jnp.float32))

        for j in range(NQ):
            wait_recv(agR.at[2, j], rsR.at[N_HOPS - 1, j])
            wr((my - 2) % N_DEV, j, 0, agR[2, j].astype(jnp.float32))
            wait_recv(agL.at[2, j], rsL.at[N_HOPS - 1, j])
            wr((my + 2) % N_DEV, j, 1, agL[2, j].astype(jnp.float32))
        for r in pending:
            r.wait_send()

    return pl.pallas_call(
        body,
        out_shape=jax.ShapeDtypeStruct((m, d), jnp.float32),
        in_specs=[
            pl.BlockSpec(memory_space=pltpu.VMEM),
            pl.BlockSpec(memory_space=pltpu.VMEM),
        ],
        out_specs=pl.BlockSpec(memory_space=pltpu.VMEM),
        scratch_shapes=[
            pltpu.VMEM((N_RS, NQ, chunk, qw), WIRE_DT),
            pltpu.VMEM((N_RS, NQ, chunk, qw), WIRE_DT),
            pltpu.VMEM((N_DEV, NQ, chunk, qw), WIRE_DT),
            pltpu.VMEM((N_RS, NQ, chunk, qw), WIRE_DT),
            pltpu.VMEM((N_RS, NQ, chunk, qw), WIRE_DT),
            pltpu.VMEM((N_DEV, NQ, chunk, qw), WIRE_DT),
            pltpu.SemaphoreType.DMA((N_HOPS, NQ)),
            pltpu.SemaphoreType.DMA((N_HOPS, NQ)),
            pltpu.SemaphoreType.DMA((N_HOPS, NQ)),
            pltpu.SemaphoreType.DMA((N_HOPS, NQ)),
        ],
        compiler_params=pltpu.CompilerParams(
            collective_id=None if _KMODE == "gemm" else 0
        ),
    )(dy, W)


# device time: 41576 ns/iter; 2.4900x vs baseline; 1.0216x over previous
import os

import jax
import jax.numpy as jnp
from jax import lax
from jax.experimental import pallas as pl
from jax.experimental.pallas import tpu as pltpu

_KMODE = os.environ.get("KMODE", "full")

N_DEV = 4
N_RS = N_DEV - 1
N_HOPS = 2 * N_RS
NQ = int(os.environ.get("KNQ", "2"))
WIRE_DT = jnp.bfloat16


def kernel(dy, W):
    m, k = dy.shape
    d = W.shape[0]
    chunk = m // N_DEV
    half = d // 2
    qw = half // NQ

    def body(
        dy_ref, w_ref, out_ref,
        sbufR, rbufR, agR, sbufL, rbufL, agL,
        ssR, rsR, ssL, rsL,
    ):
        my = lax.axis_index("i")
        left = (my - 1) % N_DEV
        right = (my + 1) % N_DEV

        if _KMODE != "gemm":
            barrier_sem = pltpu.get_barrier_semaphore()
            for nbr in (left, right):
                pl.semaphore_signal(
                    barrier_sem, inc=1,
                    device_id=(nbr,), device_id_type=pl.DeviceIdType.MESH,
                )

        def compute(c, c0, w):
            if _KMODE in ("comm", "noadd"):
                return
            out_ref[pl.ds(c * chunk, chunk), pl.ds(c0, w)] = lax.dot_general(
                dy_ref[pl.ds(c * chunk, chunk), :].astype(jnp.bfloat16),
                w_ref[pl.ds(c0, w), :].astype(jnp.bfloat16),
                dimension_numbers=(((1,), (1,)), ((), ())),
                preferred_element_type=jnp.float32,
            )

        pending = []

        def rd(c, j, dirn):
            c0 = j * qw if dirn == 0 else half + j * qw
            return out_ref[pl.ds(c * chunk, chunk), pl.ds(c0, qw)]

        def wr(c, j, dirn, val):
            c0 = j * qw if dirn == 0 else half + j * qw
            out_ref[pl.ds(c * chunk, chunk), pl.ds(c0, qw)] = val

        def send(src, dst, send_sem, recv_sem, dev):
            r = pltpu.make_async_remote_copy(
                src_ref=src, dst_ref=dst, send_sem=send_sem,
                recv_sem=recv_sem, device_id=(dev,),
                device_id_type=pl.DeviceIdType.MESH,
            )
            r.start()
            pending.append(r)

        def wait_recv(dst, sem):
            pltpu.make_async_remote_copy(
                src_ref=dst, dst_ref=dst, send_sem=sem, recv_sem=sem,
                device_id=(left,), device_id_type=pl.DeviceIdType.MESH,
            ).wait_recv()

        def send_rs(dirn, s, j):
            if dirn == 0:
                sbufR[s, j] = rd((my - s) % N_DEV, j, 0).astype(WIRE_DT)
                send(sbufR.at[s, j], rbufR.at[s, j],
                     ssR.at[s, j], rsR.at[s, j], right)
            else:
                sbufL[s, j] = rd((my + s) % N_DEV, j, 1).astype(WIRE_DT)
                send(sbufL.at[s, j], rbufL.at[s, j],
                     ssL.at[s, j], rsL.at[s, j], left)

        def recv_rs(dirn, s, j):
            if dirn == 0:
                wait_recv(rbufR.at[s, j], rsR.at[s, j])
                if _KMODE == "noadd":
                    return
                rc = (my - s - 1) % N_DEV
                wr(rc, j, 0, rd(rc, j, 0) + rbufR[s, j].astype(jnp.float32))
            else:
                wait_recv(rbufL.at[s, j], rsL.at[s, j])
                if _KMODE == "noadd":
                    return
                rc = (my + s + 1) % N_DEV
                wr(rc, j, 1, rd(rc, j, 1) + rbufL[s, j].astype(jnp.float32))

        if _KMODE == "gemm":
            for c in range(N_DEV):
                compute((my + c) % N_DEV, 0, d)
            return

        compute(my, 0, half)
        pl.semaphore_wait(barrier_sem, 2)
        for j in range(NQ):
            send_rs(0, 0, j)
        compute(my, half, half)
        for j in range(NQ):
            send_rs(1, 0, j)

        compute((my - 1) % N_DEV, 0, half)
        compute((my + 1) % N_DEV, half, half)

        for j in range(NQ):
            recv_rs(0, 0, j)
            send_rs(0, 1, j)
        for j in range(NQ):
            recv_rs(1, 0, j)
            send_rs(1, 1, j)

        compute((my + 2) % N_DEV, 0, d)

        for j in range(NQ):
            recv_rs(0, 1, j)
            send_rs(0, 2, j)
        for j in range(NQ):
            recv_rs(1, 1, j)
            send_rs(1, 2, j)

        compute((my + 1) % N_DEV, 0, half)
        compute((my - 1) % N_DEV, half, half)

        for j in range(NQ):
            recv_rs(0, 2, j)
            rc = (my + 1) % N_DEV
            agR[3, j] = rd(rc, j, 0).astype(WIRE_DT)
            send(agR.at[3, j], agR.at[0, j],
                 ssR.at[N_RS, j], rsR.at[N_RS, j], right)
        for j in range(NQ):
            recv_rs(1, 2, j)
            rc = (my - 1) % N_DEV
            agL[3, j] = rd(rc, j, 1).astype(WIRE_DT)
            send(agL.at[3, j], agL.at[0, j],
                 ssL.at[N_RS, j], rsL.at[N_RS, j], left)

        for t in (1, 2):
            h = N_RS + t
            for j in range(NQ):
                wait_recv(agR.at[t - 1, j], rsR.at[h - 1, j])
                send(agR.at[t - 1, j], agR.at[t, j],
                     ssR.at[h, j], rsR.at[h, j], right)
                wr((my - t + 1) % N_DEV, j, 0,
                   agR[t - 1, j].astype(jnp.float32))
            for j in range(NQ):
                wait_recv(agL.at[t - 1, j], rsL.at[h - 1, j])
                send(agL.at[t - 1, j], agL.at[t, j],
                     ssL.at[h, j], rsL.at[h, j], left)
                wr((my + t - 1) % N_DEV, j, 1,
                   agL[t - 1, j].astype(jnp.float32))

        for j in range(NQ):
            wait_recv(agR.at[2, j], rsR.at[N_HOPS - 1, j])
            wr((my - 2) % N_DEV, j, 0, agR[2, j].astype(jnp.float32))
            wait_recv(agL.at[2, j], rsL.at[N_HOPS - 1, j])
            wr((my + 2) % N_DEV, j, 1, agL[2, j].astype(jnp.float32))
        for r in pending:
            r.wait_send()

    return pl.pallas_call(
        body,
        out_shape=jax.ShapeDtypeStruct((m, d), jnp.float32),
        in_specs=[
            pl.BlockSpec(memory_space=pltpu.VMEM),
            pl.BlockSpec(memory_space=pltpu.VMEM),
        ],
        out_specs=pl.BlockSpec(memory_space=pltpu.VMEM),
        scratch_shapes=[
            pltpu.VMEM((N_RS, NQ, chunk, qw), WIRE_DT),
            pltpu.VMEM((N_RS, NQ, chunk, qw), WIRE_DT),
            pltpu.VMEM((N_DEV, NQ, chunk, qw), WIRE_DT),
            pltpu.VMEM((N_RS, NQ, chunk, qw), WIRE_DT),
            pltpu.VMEM((N_RS, NQ, chunk, qw), WIRE_DT),
            pltpu.VMEM((N_DEV, NQ, chunk, qw), WIRE_DT),
            pltpu.SemaphoreType.DMA((N_HOPS, NQ)),
            pltpu.SemaphoreType.DMA((N_HOPS, NQ)),
            pltpu.SemaphoreType.DMA((N_HOPS, NQ)),
            pltpu.SemaphoreType.DMA((N_HOPS, NQ)),
        ],
        compiler_params=pltpu.CompilerParams(
            collective_id=None if _KMODE == "gemm" else 0
        ),
    )(dy, W)
